# Initial kernel scaffold; baseline (speedup 1.0000x reference)
import jax
import jax.numpy as jnp
from jax.experimental import pallas as pl


def kernel(frame_colors, frame_selection, split_ratios, temperature):
    raise NotImplementedError("write your pallas kernel here")


# speedup vs baseline: 75.7809x; 75.7809x over previous
import jax
import jax.numpy as jnp
from jax.experimental import pallas as pl

_F = 16
_C = 3
_R = _F * _C
_DEPTH = 5
_OUT_HW = 256
_CELLS = 2 ** _DEPTH
_PIX = _OUT_HW // _CELLS

_f32 = jnp.float32


def _mm(a, b):
    return jnp.dot(a, b, precision=jax.lax.Precision.HIGHEST,
                   preferred_element_type=_f32)


def _iota(shape, dim):
    return jax.lax.broadcasted_iota(jnp.int32, shape, dim)


def _softmax_rows(x):
    m = jnp.max(x, axis=1, keepdims=True)
    e = jnp.exp(x - m)
    return e / jnp.sum(e, axis=1, keepdims=True)


def _masked_topk_weights(p, k):
    cols = _iota(p.shape, 1)
    work = p
    keep = jnp.zeros(p.shape, dtype=jnp.bool_)
    for _ in range(k):
        m = jnp.max(work, axis=1, keepdims=True)
        cand = work == m
        first = jnp.min(jnp.where(cand, cols, p.shape[1]), axis=1,
                        keepdims=True)
        hit = cols == first
        keep = jnp.logical_or(keep, hit)
        work = jnp.where(hit, -1.0, work)
    return jnp.where(jnp.logical_and(keep, p > 0.001), p, 0.0)


def _fractal_body(colors_ref, fl_ref, fr_ref, temp_ref, out_ref):
    t = temp_ref[...]
    p_l = _softmax_rows(fl_ref[...] / t)
    p_r = _softmax_rows(fr_ref[...] / t)
    w_l = {k: _masked_topk_weights(p_l, k) for k in (1, 2, 3)}
    w_r = {k: _masked_topk_weights(p_r, k) for k in (1, 2, 3)}

    x = jax.nn.sigmoid(colors_ref[...])

    rows = _iota((_R, 1), 0)
    is_even = ((rows // _C) % 2) == 0

    for depth in (4, 3, 2, 1, 0):
        k = 3 if depth < 2 else (2 if depth < 4 else 1)
        rp = 2 ** (_DEPTH - depth)
        rc = rp // 2
        bl = _mm(w_l[k], x)
        br = _mm(w_r[k], x)
        i_id = _iota((rc * rc, rp * rp), 0)
        o_id = _iota((rc * rc, rp * rp), 1)
        oy, ox = o_id // rp, o_id % rp
        iy, ix = i_id // rc, i_id % rc
        t_et = ((oy < rc) & (iy == oy) & (ix == ox // 2)).astype(_f32)
        t_eb = ((oy >= rc) & (iy == oy - rc) & (ix == ox // 2)).astype(_f32)
        t_ol = ((ox < rc) & (ix == ox) & (iy == oy // 2)).astype(_f32)
        t_or = ((ox >= rc) & (ix == ox - rc) & (iy == oy // 2)).astype(_f32)
        even = _mm(bl, t_et) + _mm(br, t_eb)
        odd = _mm(bl, t_ol) + _mm(br, t_or)
        x = jnp.where(is_even, even, odd)

    n = _CELLS * _CELLS
    lsel = (_iota((_CELLS, n), 1) // _CELLS == _iota((_CELLS, n), 0))
    csel = (_iota((n, _CELLS), 0) % _CELLS == _iota((n, _CELLS), 1))
    u8 = (_iota((_OUT_HW, _CELLS), 1) == _iota((_OUT_HW, _CELLS), 0) // _PIX)
    u8t = (_iota((_CELLS, _OUT_HW), 0) == _iota((_CELLS, _OUT_HW), 1) // _PIX)
    lsel, csel = lsel.astype(_f32), csel.astype(_f32)
    u8, u8t = u8.astype(_f32), u8t.astype(_f32)
    for c in range(_C):
        v = x[c:c + 1, :]
        cell = _mm(lsel * v, csel)
        out_ref[c] = _mm(u8, _mm(cell, u8t))


def _run(colors48, fl_e, fr_e, temp11):
    return pl.pallas_call(
        _fractal_body,
        out_shape=jax.ShapeDtypeStruct((_C, _OUT_HW, _OUT_HW), _f32),
    )(colors48, fl_e, fr_e, temp11)


def kernel(frame_colors, frame_selection, split_ratios, temperature):
    del split_ratios
    fl = frame_selection[:, 0, :].astype(_f32)
    fr = frame_selection[:, 1, :].astype(_f32)
    eye = jnp.eye(_C, dtype=_f32)
    on = jnp.kron(jnp.ones((_F, _F), _f32), eye) > 0
    fl_e = jnp.where(on, jnp.kron(fl, eye), -jnp.inf)
    fr_e = jnp.where(on, jnp.kron(fr, eye), -jnp.inf)
    colors48 = frame_colors.astype(_f32).reshape(_R, 1)
    temp11 = jnp.asarray(temperature, _f32).reshape(1, 1)
    return _run(colors48, fl_e, fr_e, temp11)


# speedup vs baseline: 90.8551x; 1.1989x over previous
import jax
import jax.numpy as jnp
from jax.experimental import pallas as pl

_F = 16
_C = 3
_R = _F * _C
_DEPTH = 5
_OUT_HW = 256
_CELLS = 2 ** _DEPTH
_PIX = _OUT_HW // _CELLS

_f32 = jnp.float32


def _mm(a, b):
    return jnp.dot(a, b, precision=jax.lax.Precision.HIGHEST,
                   preferred_element_type=_f32)


def _iota(shape, dim):
    return jax.lax.broadcasted_iota(jnp.int32, shape, dim)


def _softmax_rows(x):
    m = jnp.max(x, axis=1, keepdims=True)
    e = jnp.exp(x - m)
    return e / jnp.sum(e, axis=1, keepdims=True)


def _masked_topk_weights(p):
    cols = _iota(p.shape, 1)
    work = p
    keep = jnp.zeros(p.shape, dtype=jnp.bool_)
    thresh = p > 0.001
    out = {}
    for k in (1, 2, 3):
        m = jnp.max(work, axis=1, keepdims=True)
        cand = work == m
        first = jnp.min(jnp.where(cand, cols, p.shape[1]), axis=1,
                        keepdims=True)
        hit = cols == first
        keep = jnp.logical_or(keep, hit)
        work = jnp.where(hit, -1.0, work)
        out[k] = jnp.where(jnp.logical_and(keep, thresh), p, 0.0)
    return out


def _fractal_body(colors_ref, fl_ref, fr_ref, temp_ref, out_ref):
    t = temp_ref[...]
    p_l = _softmax_rows(fl_ref[...] / t)
    p_r = _softmax_rows(fr_ref[...] / t)
    w_l = _masked_topk_weights(p_l)
    w_r = _masked_topk_weights(p_r)

    x = jax.nn.sigmoid(colors_ref[...])

    rows = _iota((_R, 1), 0)
    is_even = ((rows // _C) % 2) == 0

    for depth in (4, 3, 2, 1):
        k = 3 if depth < 2 else (2 if depth < 4 else 1)
        rp = 2 ** (_DEPTH - depth)
        rc = rp // 2
        bl = _mm(w_l[k], x)
        br = _mm(w_r[k], x)
        i_id = _iota((rc * rc, rp * rp), 0)
        o_id = _iota((rc * rc, rp * rp), 1)
        oy, ox = o_id // rp, o_id % rp
        iy, ix = i_id // rc, i_id % rc
        t_et = ((oy < rc) & (iy == oy) & (ix == ox // 2)).astype(_f32)
        t_eb = ((oy >= rc) & (iy == oy - rc) & (ix == ox // 2)).astype(_f32)
        t_ol = ((ox < rc) & (ix == ox) & (iy == oy // 2)).astype(_f32)
        t_or = ((ox >= rc) & (ix == ox - rc) & (iy == oy // 2)).astype(_f32)
        even = _mm(bl, t_et) + _mm(br, t_eb)
        odd = _mm(bl, t_ol) + _mm(br, t_or)
        x = jnp.where(is_even, even, odd)

    n1 = _CELLS // 2
    hh = _OUT_HW // 2
    bl0 = _mm(w_l[3][0:8], x)
    br0 = _mm(w_r[3][0:8], x)
    l16 = (_iota((n1, n1 * n1), 1) // n1 == _iota((n1, n1 * n1), 0))
    c16 = (_iota((n1 * n1, n1), 0) % n1 == _iota((n1 * n1, n1), 1))
    u8r = (_iota((hh, n1), 1) == _iota((hh, n1), 0) // _PIX)
    u16t = (_iota((n1, _OUT_HW), 0) == _iota((n1, _OUT_HW), 1) // (2 * _PIX))
    l16, c16 = l16.astype(_f32), c16.astype(_f32)
    u8r, u16t = u8r.astype(_f32), u16t.astype(_f32)
    for c in range(_C):
        cell_t = _mm(l16 * bl0[c:c + 1, :], c16)
        cell_b = _mm(l16 * br0[c:c + 1, :], c16)
        out_ref[c, 0:hh, :] = _mm(u8r, _mm(cell_t, u16t))
        out_ref[c, hh:_OUT_HW, :] = _mm(u8r, _mm(cell_b, u16t))


def _run(colors48, fl_e, fr_e, temp11):
    return pl.pallas_call(
        _fractal_body,
        out_shape=jax.ShapeDtypeStruct((_C, _OUT_HW, _OUT_HW), _f32),
    )(colors48, fl_e, fr_e, temp11)


def kernel(frame_colors, frame_selection, split_ratios, temperature):
    del split_ratios
    fl = frame_selection[:, 0, :].astype(_f32)
    fr = frame_selection[:, 1, :].astype(_f32)
    eye = jnp.eye(_C, dtype=_f32)
    on = jnp.kron(jnp.ones((_F, _F), _f32), eye) > 0
    fl_e = jnp.where(on, jnp.kron(fl, eye), -jnp.inf)
    fr_e = jnp.where(on, jnp.kron(fr, eye), -jnp.inf)
    colors48 = frame_colors.astype(_f32).reshape(_R, 1)
    temp11 = jnp.asarray(temperature, _f32).reshape(1, 1)
    return _run(colors48, fl_e, fr_e, temp11)


# speedup vs baseline: 100.3446x; 1.1044x over previous
import jax
import jax.numpy as jnp
from jax.experimental import pallas as pl

_F = 16
_C = 3
_R = _F * _C
_DEPTH = 5
_OUT_HW = 256
_CELLS = 2 ** _DEPTH
_PIX = _OUT_HW // _CELLS

_f32 = jnp.float32


def _mm(a, b):
    return jnp.dot(a, b, precision=jax.lax.Precision.HIGHEST,
                   preferred_element_type=_f32)


def _iota(shape, dim):
    return jax.lax.broadcasted_iota(jnp.int32, shape, dim)


def _softmax_rows(x):
    m = jnp.max(x, axis=1, keepdims=True)
    e = jnp.exp(x - m)
    return e / jnp.sum(e, axis=1, keepdims=True)


def _masked_topk_weights(p):
    cols = _iota(p.shape, 1)
    work = p
    keep = jnp.zeros(p.shape, dtype=jnp.bool_)
    thresh = p > 0.001
    out = {}
    for k in (1, 2, 3):
        m = jnp.max(work, axis=1, keepdims=True)
        cand = work == m
        first = jnp.min(jnp.where(cand, cols, p.shape[1]), axis=1,
                        keepdims=True)
        hit = cols == first
        keep = jnp.logical_or(keep, hit)
        work = jnp.where(hit, -1.0, work)
        out[k] = jnp.where(jnp.logical_and(keep, thresh), p, 0.0)
    return out


def _fractal_body(colors_ref, sel_ref, temp_ref, out_ref):
    t = temp_ref[...]
    probs = _softmax_rows(sel_ref[...] / t)
    w = _masked_topk_weights(probs)

    r48 = _iota((_R, 32), 0) // _C
    a_l = (_iota((_R, 32), 1) == 2 * r48).astype(_f32)
    a_r = (_iota((_R, 32), 1) == 2 * r48 + 1).astype(_f32)
    b = (_iota((_F, _R), 0) == _iota((_F, _R), 1) // _C).astype(_f32)
    m_ch = (_iota((_R, _R), 0) % _C == _iota((_R, _R), 1) % _C).astype(_f32)
    w_l = {k: _mm(_mm(a_l, w[k]), b) * m_ch for k in (1, 2, 3)}
    w_r = {k: _mm(_mm(a_r, w[k]), b) * m_ch for k in (1, 2, 3)}

    sig = jax.nn.sigmoid(colors_ref[...])
    a48 = (_iota((_R, _F), 1) == _iota((_R, _F), 0) // _C).astype(_f32)
    k_ch = (_iota((_R, _C), 1) == _iota((_R, _C), 0) % _C).astype(_f32)
    x = _mm(_mm(a48, sig) * k_ch, jnp.ones((_C, 1), _f32))

    rows = _iota((_R, 1), 0)
    is_even = ((rows // _C) % 2) == 0

    for depth in (4, 3, 2, 1):
        k = 3 if depth < 2 else (2 if depth < 4 else 1)
        rp = 2 ** (_DEPTH - depth)
        rc = rp // 2
        bl = _mm(w_l[k], x)
        br = _mm(w_r[k], x)
        i_id = _iota((rc * rc, rp * rp), 0)
        o_id = _iota((rc * rc, rp * rp), 1)
        oy, ox = o_id // rp, o_id % rp
        iy, ix = i_id // rc, i_id % rc
        t_et = ((oy < rc) & (iy == oy) & (ix == ox // 2)).astype(_f32)
        t_eb = ((oy >= rc) & (iy == oy - rc) & (ix == ox // 2)).astype(_f32)
        t_ol = ((ox < rc) & (ix == ox) & (iy == oy // 2)).astype(_f32)
        t_or = ((ox >= rc) & (ix == ox - rc) & (iy == oy // 2)).astype(_f32)
        even = _mm(bl, t_et) + _mm(br, t_eb)
        odd = _mm(bl, t_ol) + _mm(br, t_or)
        x = jnp.where(is_even, even, odd)

    n1 = _CELLS // 2
    hh = _OUT_HW // 2
    bl0 = _mm(w_l[3][0:8], x)
    br0 = _mm(w_r[3][0:8], x)
    l16 = (_iota((n1, n1 * n1), 1) // n1 == _iota((n1, n1 * n1), 0))
    c16 = (_iota((n1 * n1, n1), 0) % n1 == _iota((n1 * n1, n1), 1))
    u8r = (_iota((hh, n1), 1) == _iota((hh, n1), 0) // _PIX)
    u16t = (_iota((n1, _OUT_HW), 0) == _iota((n1, _OUT_HW), 1) // (2 * _PIX))
    l16, c16 = l16.astype(_f32), c16.astype(_f32)
    u8r, u16t = u8r.astype(_f32), u16t.astype(_f32)
    for c in range(_C):
        cell_t = _mm(l16 * bl0[c:c + 1, :], c16)
        cell_b = _mm(l16 * br0[c:c + 1, :], c16)
        out_ref[c, 0:hh, :] = _mm(u8r, _mm(cell_t, u16t))
        out_ref[c, hh:_OUT_HW, :] = _mm(u8r, _mm(cell_b, u16t))


def _run(colors, sel32, temp11):
    return pl.pallas_call(
        _fractal_body,
        out_shape=jax.ShapeDtypeStruct((_C, _OUT_HW, _OUT_HW), _f32),
    )(colors, sel32, temp11)


def kernel(frame_colors, frame_selection, split_ratios, temperature):
    del split_ratios
    sel32 = frame_selection.astype(_f32).reshape(2 * _F, _F)
    temp11 = jnp.asarray(temperature, _f32).reshape(1, 1)
    return _run(frame_colors.astype(_f32), sel32, temp11)


# speedup vs baseline: 113.3322x; 1.1294x over previous
import jax
import jax.numpy as jnp
from jax.experimental import pallas as pl

_F = 16
_C = 3
_R = _F * _C
_DEPTH = 5
_OUT_HW = 256
_CELLS = 2 ** _DEPTH
_PIX = _OUT_HW // _CELLS

_f32 = jnp.float32
_bf16 = jnp.bfloat16


def _mm(a, b):
    return jnp.dot(a, b, precision=jax.lax.Precision.HIGHEST,
                   preferred_element_type=_f32)


def _mm1(a, b):
    return jnp.dot(a, b, preferred_element_type=_f32)


def _split(x):
    hi = x.astype(_bf16)
    lo = (x - hi.astype(_f32)).astype(_bf16)
    return hi, lo


def _mm_db(x, t):
    hi, lo = _split(x)
    return _mm1(hi, t) + _mm1(lo, t)


def _mm_bd(t, x):
    hi, lo = _split(x)
    return _mm1(t, hi) + _mm1(t, lo)


def _iota(shape, dim):
    return jax.lax.broadcasted_iota(jnp.int32, shape, dim)


def _softmax_rows(x):
    m = jnp.max(x, axis=1, keepdims=True)
    e = jnp.exp(x - m)
    return e / jnp.sum(e, axis=1, keepdims=True)


def _masked_topk_weights(p):
    cols = _iota(p.shape, 1)
    work = p
    keep = jnp.zeros(p.shape, dtype=jnp.bool_)
    thresh = p > 0.001
    out = {}
    for k in (1, 2, 3):
        m = jnp.max(work, axis=1, keepdims=True)
        cand = work == m
        first = jnp.min(jnp.where(cand, cols, p.shape[1]), axis=1,
                        keepdims=True)
        hit = cols == first
        keep = jnp.logical_or(keep, hit)
        work = jnp.where(hit, -1.0, work)
        out[k] = jnp.where(jnp.logical_and(keep, thresh), p, 0.0)
    return out


def _fractal_body(colors_ref, sel_ref, temp_ref, out_ref):
    t = temp_ref[...]
    probs = _softmax_rows(sel_ref[...] / t)
    w = _masked_topk_weights(probs)

    r48 = _iota((_R, 32), 0) // _C
    a_l = (_iota((_R, 32), 1) == 2 * r48).astype(_f32)
    a_r = (_iota((_R, 32), 1) == 2 * r48 + 1).astype(_f32)
    b = (_iota((_F, _R), 0) == _iota((_F, _R), 1) // _C).astype(_f32)
    m_ch = (_iota((_R, _R), 0) % _C == _iota((_R, _R), 1) % _C).astype(_f32)
    w_l = {k: _mm(_mm(a_l, w[k]), b) * m_ch for k in (1, 2, 3)}
    w_r = {k: _mm(_mm(a_r, w[k]), b) * m_ch for k in (1, 2, 3)}

    sig = jax.nn.sigmoid(colors_ref[...])
    a48 = (_iota((_R, _F), 1) == _iota((_R, _F), 0) // _C).astype(_f32)
    k_ch = (_iota((_R, _C), 1) == _iota((_R, _C), 0) % _C).astype(_f32)
    x = _mm(_mm(a48, sig) * k_ch, jnp.ones((_C, 1), _f32))

    rows = _iota((_R, 1), 0)
    is_even = ((rows // _C) % 2) == 0

    for depth in (4, 3, 2, 1):
        k = 3 if depth < 2 else (2 if depth < 4 else 1)
        rp = 2 ** (_DEPTH - depth)
        rc = rp // 2
        bl = _mm(w_l[k], x)
        br = _mm(w_r[k], x)
        i_id = _iota((rc * rc, rp * rp), 0)
        o_id = _iota((rc * rc, rp * rp), 1)
        oy, ox = o_id // rp, o_id % rp
        iy, ix = i_id // rc, i_id % rc
        t_et = ((oy < rc) & (iy == oy) & (ix == ox // 2)).astype(_bf16)
        t_eb = ((oy >= rc) & (iy == oy - rc) & (ix == ox // 2)).astype(_bf16)
        t_ol = ((ox < rc) & (ix == ox) & (iy == oy // 2)).astype(_bf16)
        t_or = ((ox >= rc) & (ix == ox - rc) & (iy == oy // 2)).astype(_bf16)
        bl_h, bl_l = _split(bl)
        br_h, br_l = _split(br)
        even = (_mm1(bl_h, t_et) + _mm1(bl_l, t_et)
                + _mm1(br_h, t_eb) + _mm1(br_l, t_eb))
        odd = (_mm1(bl_h, t_ol) + _mm1(bl_l, t_ol)
               + _mm1(br_h, t_or) + _mm1(br_l, t_or))
        x = jnp.where(is_even, even, odd)

    n1 = _CELLS // 2
    hh = _OUT_HW // 2
    bl0 = _mm(w_l[3][0:8], x)
    br0 = _mm(w_r[3][0:8], x)
    l16 = (_iota((n1, n1 * n1), 1) // n1
           == _iota((n1, n1 * n1), 0)).astype(_f32)
    c16 = (_iota((n1 * n1, n1), 0) % n1
           == _iota((n1 * n1, n1), 1)).astype(_bf16)
    u8r = (_iota((hh, n1), 1) == _iota((hh, n1), 0) // _PIX).astype(_bf16)
    u16t = (_iota((n1, _OUT_HW), 0)
            == _iota((n1, _OUT_HW), 1) // (2 * _PIX)).astype(_bf16)
    for c in range(_C):
        cell_t = _mm_db(l16 * bl0[c:c + 1, :], c16)
        cell_b = _mm_db(l16 * br0[c:c + 1, :], c16)
        out_ref[c, 0:hh, :] = _mm_bd(u8r, _mm_db(cell_t, u16t))
        out_ref[c, hh:_OUT_HW, :] = _mm_bd(u8r, _mm_db(cell_b, u16t))


def _run(colors, sel32, temp11):
    return pl.pallas_call(
        _fractal_body,
        out_shape=jax.ShapeDtypeStruct((_C, _OUT_HW, _OUT_HW), _f32),
    )(colors, sel32, temp11)


def kernel(frame_colors, frame_selection, split_ratios, temperature):
    del split_ratios
    sel32 = frame_selection.astype(_f32).reshape(2 * _F, _F)
    temp11 = jnp.asarray(temperature, _f32).reshape(1, 1)
    return _run(frame_colors.astype(_f32), sel32, temp11)
